# baseline (speedup 1.0000x reference)
import dataclasses
import functools

import jax
import jax.numpy as jnp
from jax import lax
from jax.experimental import pallas as pl
from jax.experimental.pallas import tpu as pltpu
from jax.experimental.pallas import tpu_sc as plsc

N_USER = 10000
N_ITEM = 40000
N = N_USER + N_ITEM
E = 800000
D = 64
B = 1024

NC = 2
NS = 16
NW = NC * NS
L = 16

HALF = N // 2
ACC_ROWS = 25024
TRASH = 25008
ZROWS = 1564
WIN = 128
WSTEP = 120

_mesh = plsc.VectorSubcoreMesh(core_axis_name="c", subcore_axis_name="s")

_sc_params = pltpu.CompilerParams()
for _f, _v in (("needs_layout_passes", False), ("use_tc_tiling_on_sc", False)):
    if _f in pltpu.CompilerParams.__dataclass_fields__:
        _sc_params = dataclasses.replace(_sc_params, **{_f: _v})


_GDN = lax.GatherDimensionNumbers(
    offset_dims=(), collapsed_slice_dims=(0,), start_index_map=(0,))


def _bcast_lane(vec, j):
    idx = jnp.full((L, 1), j, jnp.int32)
    return lax.gather(vec, idx, _GDN, (1,),
                      mode=lax.GatherScatterMode.PROMISE_IN_BOUNDS)


@functools.partial(
    pl.kernel,
    mesh=_mesh,
    compiler_params=_sc_params,
    out_type=jax.ShapeDtypeStruct((N, D), jnp.float32),
    scratch_types=[
        pltpu.VMEM((3, WIN), jnp.int32),
        pltpu.VMEM((3, WIN), jnp.int32),
        pltpu.VMEM((3, WIN), jnp.float32),
        pltpu.VMEM((3, WIN), jnp.int32),
        pltpu.VMEM((3, WIN, D), jnp.float32),
        pltpu.VMEM((L,), jnp.int32),
        pltpu.VMEM_SHARED((ACC_ROWS, D), jnp.float32),
        pltpu.SemaphoreType.DMA,
        pltpu.SemaphoreType.DMA,
        pltpu.SemaphoreType.DMA,
    ],
)
def _spmm(ego_hbm, col_hbm, row_hbm, val_hbm, out_hbm,
          colv, rowv, valv, rowlv, gbuf, bsv, acc, sem, sem_i, sem_s):
    core = lax.axis_index("c")
    sub = lax.axis_index("s")
    iot = lax.iota(jnp.int32, L)
    zvec = jnp.zeros((L,), jnp.float32)

    @pl.loop(0, WIN)
    def _(r):
        for q in range(D // L):
            gbuf[0, r, pl.ds(q * L, L)] = zvec

    zbase = sub * ZROWS

    @pl.loop(0, ZROWS // WIN)
    def _(i):
        pltpu.async_copy(gbuf.at[0], acc.at[pl.ds(zbase + i * WIN, WIN)], sem_i)

    rem = ZROWS % WIN
    if rem:
        pltpu.async_copy(gbuf.at[0, pl.ds(0, rem)],
                         acc.at[pl.ds(zbase + (ZROWS // WIN) * WIN, rem)], sem_i)

    def _bs_body(_, lohi):
        lo, hi = lohi
        mid = (lo + hi) // 2
        a16 = (mid // L) * L
        pltpu.sync_copy(row_hbm.at[pl.ds(a16, L)], bsv)
        probe = bsv[...]
        x = jnp.sum(jnp.where(iot == (mid - a16), probe, jnp.zeros_like(probe)))
        take_hi = x >= HALF
        return (jnp.where(take_hi, lo, mid + 1), jnp.where(take_hi, mid, hi))

    m, _ = lax.fori_loop(0, 20, _bs_body, (jnp.int32(0), jnp.int32(E)))

    @pl.loop(0, ZROWS // WIN)
    def _(i):
        pltpu.make_async_copy(gbuf.at[0],
                              acc.at[pl.ds(zbase + i * WIN, WIN)], sem_i).wait()
    if ZROWS % WIN:
        pltpu.make_async_copy(
            gbuf.at[0, pl.ds(0, ZROWS % WIN)],
            acc.at[pl.ds(zbase + (ZROWS // WIN) * WIN, ZROWS % WIN)],
            sem_i).wait()

    lo_c = jnp.where(core == 0, 0, m)
    hi_c = jnp.where(core == 0, m, E)
    cnt = hi_c - lo_c
    b_start = lo_c + (cnt * sub) // NS
    b_end = lo_c + (cnt * (sub + 1)) // NS
    row_base = core * HALF

    plsc.subcore_barrier()

    nwin = jnp.maximum((b_end - b_start + (WSTEP - 1)) // WSTEP, 1)

    def _astart(w):
        return jnp.minimum(((b_start + w * WSTEP) // 8) * 8, E - WIN)

    def _issue_idx(w, b):
        a = _astart(w)
        pltpu.async_copy(col_hbm.at[pl.ds(a, WIN)], colv.at[b], sem_i)
        pltpu.async_copy(row_hbm.at[pl.ds(a, WIN)], rowv.at[b], sem_i)
        pltpu.async_copy(val_hbm.at[pl.ds(a, WIN)], valv.at[b], sem_i)

    def _wait_idx(b):
        pltpu.make_async_copy(col_hbm.at[pl.ds(0, WIN)], colv.at[b], sem_i).wait()
        pltpu.make_async_copy(row_hbm.at[pl.ds(0, WIN)], rowv.at[b], sem_i).wait()
        pltpu.make_async_copy(val_hbm.at[pl.ds(0, WIN)], valv.at[b], sem_i).wait()

    def _issue_gather(b):
        pltpu.async_copy(ego_hbm.at[colv.at[b]], gbuf.at[b], sem)

    def _wait_gather(b):
        pltpu.make_async_copy(ego_hbm.at[colv.at[b]], gbuf.at[b], sem).wait()

    for _k in range(2):
        @pl.when(_k < nwin)
        def _(k=_k):
            _issue_idx(k, k)
    for _k in range(2):
        @pl.when(_k < nwin)
        def _(k=_k):
            _wait_idx(k)
            _issue_gather(k)

    trashv = jnp.full((L,), TRASH, jnp.int32) + iot

    def _window(w, carry):
        b = w % 3
        b2 = (w + 2) % 3

        @pl.when(w + 2 < nwin)
        def _():
            _issue_idx(w + 2, b2)

        _wait_gather(b)

        @pl.when(w + 2 < nwin)
        def _():
            _wait_idx(b2)
            @pl.when(w >= 1)
            def _():
                pltpu.make_async_copy(gbuf.at[b2], acc.at[rowlv.at[b2]],
                                      sem_s).wait()
            _issue_gather(b2)

        nom = b_start + w * WSTEP
        a = _astart(w)
        vhi = jnp.minimum(nom + WSTEP, b_end)

        @plsc.parallel_loop(0, WIN // L, unroll=4)
        def _(g):
            eoff = a + g * L
            idxe = eoff + iot
            valid = (idxe >= nom) & (idxe < vhi)
            rw = rowv[b, pl.ds(g * L, L)]
            rlocal = jnp.where(valid, rw - row_base, trashv)
            rowlv[b, pl.ds(g * L, L)] = rlocal
            vw = valv[b, pl.ds(g * L, L)]
            for j in range(L):
                er = g * L + j
                bb = _bcast_lane(vw, j)
                for q in range(D // L):
                    sl = pl.ds(q * L, L)
                    gbuf[b, er, sl] = gbuf[b, er, sl] * bb

        pltpu.async_copy(gbuf.at[b], acc.at[rowlv.at[b]], sem_s, add=True)
        return carry

    lax.fori_loop(0, nwin, _window, 0)

    @pl.when(nwin > 2)
    def _():
        pltpu.make_async_copy(gbuf.at[0], acc.at[rowlv.at[0]], sem_s).wait()

    @pl.when(nwin > 1)
    def _():
        pltpu.make_async_copy(gbuf.at[0], acc.at[rowlv.at[0]], sem_s).wait()

    pltpu.make_async_copy(gbuf.at[0], acc.at[rowlv.at[0]], sem_s).wait()

    plsc.subcore_barrier()

    main = (HALF // NS // 8) * 8
    pltpu.sync_copy(acc.at[pl.ds(sub * main, main)],
                    out_hbm.at[pl.ds(row_base + sub * main, main)])
    tail = HALF - NS * main

    @pl.when(sub < tail)
    def _():
        pltpu.sync_copy(acc.at[pl.ds(NS * main + sub, 1)],
                        out_hbm.at[pl.ds(row_base + NS * main + sub, 1)])


def _dense_body(side_ref, ego_ref, wg_ref, bg_ref, wb_ref, bb_ref,
                ego_out_ref, norm_out_ref):
    s = side_ref[...]
    e = ego_ref[...]
    x = jnp.dot(s, wg_ref[...], preferred_element_type=jnp.float32) + bg_ref[...]
    x = x + jnp.dot(e * s, wb_ref[...], preferred_element_type=jnp.float32) + bb_ref[...]
    y = jnp.where(x >= 0, x, x * jnp.float32(0.2))
    ego_out_ref[...] = y
    nrm = jnp.sqrt(jnp.sum(y * y, axis=1, keepdims=True))
    norm_out_ref[...] = y / jnp.maximum(nrm, jnp.float32(1e-12))


_R = 2000
_dense = pl.pallas_call(
    _dense_body,
    grid=(N // _R,),
    in_specs=[
        pl.BlockSpec((_R, D), lambda i: (i, 0)),
        pl.BlockSpec((_R, D), lambda i: (i, 0)),
        pl.BlockSpec((D, D), lambda i: (0, 0)),
        pl.BlockSpec((1, D), lambda i: (0, 0)),
        pl.BlockSpec((D, D), lambda i: (0, 0)),
        pl.BlockSpec((1, D), lambda i: (0, 0)),
    ],
    out_specs=[
        pl.BlockSpec((_R, D), lambda i: (i, 0)),
        pl.BlockSpec((_R, D), lambda i: (i, 0)),
    ],
    out_shape=[
        jax.ShapeDtypeStruct((N, D), jnp.float32),
        jax.ShapeDtypeStruct((N, D), jnp.float32),
    ],
)

_GW = 3 * B // NW


@functools.partial(
    pl.kernel,
    mesh=_mesh,
    compiler_params=_sc_params,
    out_type=jax.ShapeDtypeStruct((4, 3 * B, D), jnp.float32),
    scratch_types=[
        pltpu.VMEM((_GW,), jnp.int32),
        pltpu.VMEM((_GW, D), jnp.float32),
        pltpu.SemaphoreType.DMA,
    ],
)
def _gather4(e0, e1, e2, e3, idx_hbm, out_hbm, idxv, gv, sem):
    wid = lax.axis_index("s") * NC + lax.axis_index("c")
    base = wid * _GW
    pltpu.sync_copy(idx_hbm.at[pl.ds(base, _GW)], idxv)
    for k, eref in enumerate((e0, e1, e2, e3)):
        pltpu.async_copy(eref.at[idxv], gv, sem).wait()
        pltpu.sync_copy(gv, out_hbm.at[k, pl.ds(base, _GW)])


def kernel(user_emb, item_emb,
           W_gc_0, b_gc_0, W_bi_0, b_bi_0,
           W_gc_1, b_gc_1, W_bi_1, b_bi_1,
           W_gc_2, b_gc_2, W_bi_2, b_bi_2,
           adj_row, adj_col, adj_val,
           users, pos_items, neg_items):
    ego = jnp.concatenate([user_emb, item_emb], axis=0)
    col = adj_col.astype(jnp.int32)
    row = adj_row.astype(jnp.int32)
    val = adj_val.astype(jnp.float32)

    embs = [ego]
    for (wg, bg, wb, bb) in (
        (W_gc_0, b_gc_0, W_bi_0, b_bi_0),
        (W_gc_1, b_gc_1, W_bi_1, b_bi_1),
        (W_gc_2, b_gc_2, W_bi_2, b_bi_2),
    ):
        side = _spmm(ego, col, row, val)
        ego, nrm = _dense(side, ego, wg, bg, wb, bb)
        embs.append(nrm)

    idx = jnp.concatenate([
        users.astype(jnp.int32),
        pos_items.astype(jnp.int32) + N_USER,
        neg_items.astype(jnp.int32) + N_USER,
    ])
    g = _gather4(embs[0], embs[1], embs[2], embs[3], idx)
    allg = jnp.transpose(g, (1, 0, 2)).reshape(3 * B, 4 * D)
    return (allg[:B], allg[B:2 * B], allg[2 * B:])


# speedup vs baseline: 1.0244x; 1.0244x over previous
import dataclasses
import functools

import jax
import jax.numpy as jnp
from jax import lax
from jax.experimental import pallas as pl
from jax.experimental.pallas import tpu as pltpu
from jax.experimental.pallas import tpu_sc as plsc

N_USER = 10000
N_ITEM = 40000
N = N_USER + N_ITEM
E = 800000
D = 64
B = 1024

NC = 2
NS = 16
NW = NC * NS
L = 16

HALF = N // 2
ACC_ROWS = 25024
TRASH = 25008
ZROWS = 1564
WIN = 128
WSTEP = 120

_mesh = plsc.VectorSubcoreMesh(core_axis_name="c", subcore_axis_name="s")

_sc_params = pltpu.CompilerParams()
for _f, _v in (("needs_layout_passes", False), ("use_tc_tiling_on_sc", False)):
    if _f in pltpu.CompilerParams.__dataclass_fields__:
        _sc_params = dataclasses.replace(_sc_params, **{_f: _v})


_GDN = lax.GatherDimensionNumbers(
    offset_dims=(), collapsed_slice_dims=(0,), start_index_map=(0,))


def _bcast_lane(vec, j):
    idx = jnp.full((L, 1), j, jnp.int32)
    return lax.gather(vec, idx, _GDN, (1,),
                      mode=lax.GatherScatterMode.PROMISE_IN_BOUNDS)


@functools.partial(
    pl.kernel,
    mesh=_mesh,
    compiler_params=_sc_params,
    out_type=jax.ShapeDtypeStruct((N, D), jnp.float32),
    scratch_types=[
        pltpu.VMEM((3, WIN), jnp.int32),
        pltpu.VMEM((3, WIN), jnp.int32),
        pltpu.VMEM((3, WIN), jnp.float32),
        pltpu.VMEM((3, WIN), jnp.int32),
        pltpu.VMEM((3, WIN, D), jnp.float32),
        pltpu.VMEM((L,), jnp.int32),
        pltpu.VMEM_SHARED((ACC_ROWS, D), jnp.float32),
        pltpu.SemaphoreType.DMA,
        pltpu.SemaphoreType.DMA,
        pltpu.SemaphoreType.DMA,
    ],
)
def _spmm(ego_hbm, col_hbm, row_hbm, val_hbm, out_hbm,
          colv, rowv, valv, rowlv, gbuf, bsv, acc, sem, sem_i, sem_s):
    core = lax.axis_index("c")
    sub = lax.axis_index("s")
    iot = lax.iota(jnp.int32, L)
    zvec = jnp.zeros((L,), jnp.float32)

    @pl.loop(0, WIN)
    def _(r):
        for q in range(D // L):
            gbuf[0, r, pl.ds(q * L, L)] = zvec

    zbase = sub * ZROWS

    @pl.loop(0, ZROWS // WIN)
    def _(i):
        pltpu.async_copy(gbuf.at[0], acc.at[pl.ds(zbase + i * WIN, WIN)], sem_i)

    rem = ZROWS % WIN
    if rem:
        pltpu.async_copy(gbuf.at[0, pl.ds(0, rem)],
                         acc.at[pl.ds(zbase + (ZROWS // WIN) * WIN, rem)], sem_i)

    def _bs_body(_, lohi):
        lo, hi = lohi
        mid = (lo + hi) // 2
        a16 = (mid // L) * L
        pltpu.sync_copy(row_hbm.at[pl.ds(a16, L)], bsv)
        probe = bsv[...]
        x = jnp.sum(jnp.where(iot == (mid - a16), probe, jnp.zeros_like(probe)))
        take_hi = x >= HALF
        return (jnp.where(take_hi, lo, mid + 1), jnp.where(take_hi, mid, hi))

    m, _ = lax.fori_loop(0, 20, _bs_body, (jnp.int32(0), jnp.int32(E)))

    @pl.loop(0, ZROWS // WIN)
    def _(i):
        pltpu.make_async_copy(gbuf.at[0],
                              acc.at[pl.ds(zbase + i * WIN, WIN)], sem_i).wait()
    if ZROWS % WIN:
        pltpu.make_async_copy(
            gbuf.at[0, pl.ds(0, ZROWS % WIN)],
            acc.at[pl.ds(zbase + (ZROWS // WIN) * WIN, ZROWS % WIN)],
            sem_i).wait()

    lo_c = jnp.where(core == 0, 0, m)
    hi_c = jnp.where(core == 0, m, E)
    cnt = hi_c - lo_c
    b_start = lo_c + (cnt * sub) // NS
    b_end = lo_c + (cnt * (sub + 1)) // NS
    row_base = core * HALF

    plsc.subcore_barrier()

    nwin = jnp.maximum((b_end - b_start + (WSTEP - 1)) // WSTEP, 1)

    def _astart(w):
        return jnp.minimum(((b_start + w * WSTEP) // 8) * 8, E - WIN)

    def _issue_idx(w, b):
        a = _astart(w)
        pltpu.async_copy(col_hbm.at[pl.ds(a, WIN)], colv.at[b], sem_i)
        pltpu.async_copy(row_hbm.at[pl.ds(a, WIN)], rowv.at[b], sem_i)
        pltpu.async_copy(val_hbm.at[pl.ds(a, WIN)], valv.at[b], sem_i)

    def _wait_idx(b):
        pltpu.make_async_copy(col_hbm.at[pl.ds(0, WIN)], colv.at[b], sem_i).wait()
        pltpu.make_async_copy(row_hbm.at[pl.ds(0, WIN)], rowv.at[b], sem_i).wait()
        pltpu.make_async_copy(val_hbm.at[pl.ds(0, WIN)], valv.at[b], sem_i).wait()

    def _issue_gather(b):
        pltpu.async_copy(ego_hbm.at[colv.at[b]], gbuf.at[b], sem)

    def _wait_gather(b):
        pltpu.make_async_copy(ego_hbm.at[colv.at[b]], gbuf.at[b], sem).wait()

    for _k in range(2):
        @pl.when(_k < nwin)
        def _(k=_k):
            _issue_idx(k, k)
    for _k in range(2):
        @pl.when(_k < nwin)
        def _(k=_k):
            _wait_idx(k)
            _issue_gather(k)

    trashv = jnp.full((L,), TRASH, jnp.int32) + iot

    def _window(w, carry):
        b = w % 3
        b2 = (w + 2) % 3

        @pl.when(w + 2 < nwin)
        def _():
            _issue_idx(w + 2, b2)

        _wait_gather(b)

        @pl.when(w + 2 < nwin)
        def _():
            _wait_idx(b2)
            @pl.when(w >= 1)
            def _():
                pltpu.make_async_copy(gbuf.at[b2], acc.at[rowlv.at[b2]],
                                      sem_s).wait()
            _issue_gather(b2)

        nom = b_start + w * WSTEP
        a = _astart(w)
        vhi = jnp.minimum(nom + WSTEP, b_end)

        @plsc.parallel_loop(0, WIN // L, unroll=2)
        def _(g):
            eoff = a + g * L
            idxe = eoff + iot
            valid = (idxe >= nom) & (idxe < vhi)
            rw = rowv[b, pl.ds(g * L, L)]
            rlocal = jnp.where(valid, rw - row_base, trashv)
            rowlv[b, pl.ds(g * L, L)] = rlocal
            vw = valv[b, pl.ds(g * L, L)]
            for j in range(L):
                er = g * L + j
                bb = _bcast_lane(vw, j)
                for q in range(D // L):
                    sl = pl.ds(q * L, L)
                    gbuf[b, er, sl] = gbuf[b, er, sl] * bb

        pltpu.async_copy(gbuf.at[b], acc.at[rowlv.at[b]], sem_s, add=True)
        return carry

    lax.fori_loop(0, nwin, _window, 0)

    @pl.when(nwin > 2)
    def _():
        pltpu.make_async_copy(gbuf.at[0], acc.at[rowlv.at[0]], sem_s).wait()

    @pl.when(nwin > 1)
    def _():
        pltpu.make_async_copy(gbuf.at[0], acc.at[rowlv.at[0]], sem_s).wait()

    pltpu.make_async_copy(gbuf.at[0], acc.at[rowlv.at[0]], sem_s).wait()

    plsc.subcore_barrier()

    main = (HALF // NS // 8) * 8
    pltpu.sync_copy(acc.at[pl.ds(sub * main, main)],
                    out_hbm.at[pl.ds(row_base + sub * main, main)])
    tail = HALF - NS * main

    @pl.when(sub < tail)
    def _():
        pltpu.sync_copy(acc.at[pl.ds(NS * main + sub, 1)],
                        out_hbm.at[pl.ds(row_base + NS * main + sub, 1)])


def _dense_body(side_ref, ego_ref, wg_ref, bg_ref, wb_ref, bb_ref,
                ego_out_ref, norm_out_ref):
    s = side_ref[...]
    e = ego_ref[...]
    x = jnp.dot(s, wg_ref[...], preferred_element_type=jnp.float32) + bg_ref[...]
    x = x + jnp.dot(e * s, wb_ref[...], preferred_element_type=jnp.float32) + bb_ref[...]
    y = jnp.where(x >= 0, x, x * jnp.float32(0.2))
    ego_out_ref[...] = y
    nrm = jnp.sqrt(jnp.sum(y * y, axis=1, keepdims=True))
    norm_out_ref[...] = y / jnp.maximum(nrm, jnp.float32(1e-12))


_R = 2000
_dense = pl.pallas_call(
    _dense_body,
    grid=(N // _R,),
    in_specs=[
        pl.BlockSpec((_R, D), lambda i: (i, 0)),
        pl.BlockSpec((_R, D), lambda i: (i, 0)),
        pl.BlockSpec((D, D), lambda i: (0, 0)),
        pl.BlockSpec((1, D), lambda i: (0, 0)),
        pl.BlockSpec((D, D), lambda i: (0, 0)),
        pl.BlockSpec((1, D), lambda i: (0, 0)),
    ],
    out_specs=[
        pl.BlockSpec((_R, D), lambda i: (i, 0)),
        pl.BlockSpec((_R, D), lambda i: (i, 0)),
    ],
    out_shape=[
        jax.ShapeDtypeStruct((N, D), jnp.float32),
        jax.ShapeDtypeStruct((N, D), jnp.float32),
    ],
)

_GW = 3 * B // NW


@functools.partial(
    pl.kernel,
    mesh=_mesh,
    compiler_params=_sc_params,
    out_type=jax.ShapeDtypeStruct((4, 3 * B, D), jnp.float32),
    scratch_types=[
        pltpu.VMEM((_GW,), jnp.int32),
        pltpu.VMEM((_GW, D), jnp.float32),
        pltpu.SemaphoreType.DMA,
    ],
)
def _gather4(e0, e1, e2, e3, idx_hbm, out_hbm, idxv, gv, sem):
    wid = lax.axis_index("s") * NC + lax.axis_index("c")
    base = wid * _GW
    pltpu.sync_copy(idx_hbm.at[pl.ds(base, _GW)], idxv)
    for k, eref in enumerate((e0, e1, e2, e3)):
        pltpu.async_copy(eref.at[idxv], gv, sem).wait()
        pltpu.sync_copy(gv, out_hbm.at[k, pl.ds(base, _GW)])


def kernel(user_emb, item_emb,
           W_gc_0, b_gc_0, W_bi_0, b_bi_0,
           W_gc_1, b_gc_1, W_bi_1, b_bi_1,
           W_gc_2, b_gc_2, W_bi_2, b_bi_2,
           adj_row, adj_col, adj_val,
           users, pos_items, neg_items):
    ego = jnp.concatenate([user_emb, item_emb], axis=0)
    col = adj_col.astype(jnp.int32)
    row = adj_row.astype(jnp.int32)
    val = adj_val.astype(jnp.float32)

    embs = [ego]
    for (wg, bg, wb, bb) in (
        (W_gc_0, b_gc_0, W_bi_0, b_bi_0),
        (W_gc_1, b_gc_1, W_bi_1, b_bi_1),
        (W_gc_2, b_gc_2, W_bi_2, b_bi_2),
    ):
        side = _spmm(ego, col, row, val)
        ego, nrm = _dense(side, ego, wg, bg, wb, bb)
        embs.append(nrm)

    idx = jnp.concatenate([
        users.astype(jnp.int32),
        pos_items.astype(jnp.int32) + N_USER,
        neg_items.astype(jnp.int32) + N_USER,
    ])
    g = _gather4(embs[0], embs[1], embs[2], embs[3], idx)
    allg = jnp.transpose(g, (1, 0, 2)).reshape(3 * B, 4 * D)
    return (allg[:B], allg[B:2 * B], allg[2 * B:])


# speedup vs baseline: 1.0621x; 1.0368x over previous
import dataclasses
import functools

import jax
import jax.numpy as jnp
from jax import lax
from jax.experimental import pallas as pl
from jax.experimental.pallas import tpu as pltpu
from jax.experimental.pallas import tpu_sc as plsc

N_USER = 10000
N_ITEM = 40000
N = N_USER + N_ITEM
E = 800000
D = 64
B = 1024

NC = 2
NS = 16
NW = NC * NS
L = 16

HALF = N // 2
ACC_ROWS = 25024
TRASH = 25008
ZROWS = 1564
WIN = 128
WSTEP = 120

_mesh = plsc.VectorSubcoreMesh(core_axis_name="c", subcore_axis_name="s")

_sc_params = pltpu.CompilerParams()
for _f, _v in (("needs_layout_passes", False), ("use_tc_tiling_on_sc", False)):
    if _f in pltpu.CompilerParams.__dataclass_fields__:
        _sc_params = dataclasses.replace(_sc_params, **{_f: _v})


_GDN = lax.GatherDimensionNumbers(
    offset_dims=(), collapsed_slice_dims=(0,), start_index_map=(0,))


def _bcast_lane(vec, j):
    idx = jnp.full((L, 1), j, jnp.int32)
    return lax.gather(vec, idx, _GDN, (1,),
                      mode=lax.GatherScatterMode.PROMISE_IN_BOUNDS)


@functools.partial(
    pl.kernel,
    mesh=_mesh,
    compiler_params=_sc_params,
    out_type=jax.ShapeDtypeStruct((N, D), jnp.float32),
    scratch_types=[
        pltpu.VMEM((3, WIN), jnp.int32),
        pltpu.VMEM((3, WIN), jnp.int32),
        pltpu.VMEM((3, WIN), jnp.float32),
        pltpu.VMEM((3, WIN), jnp.int32),
        pltpu.VMEM((3, WIN, D), jnp.float32),
        pltpu.VMEM((L,), jnp.int32),
        pltpu.VMEM_SHARED((ACC_ROWS, D), jnp.float32),
        pltpu.SemaphoreType.DMA,
        pltpu.SemaphoreType.DMA,
        pltpu.SemaphoreType.DMA,
    ],
)
def _spmm(ego_hbm, col_hbm, row_hbm, val_hbm, out_hbm,
          colv, rowv, valv, rowlv, gbuf, bsv, acc, sem, sem_i, sem_s):
    core = lax.axis_index("c")
    sub = lax.axis_index("s")
    iot = lax.iota(jnp.int32, L)
    zvec = jnp.zeros((L,), jnp.float32)

    @pl.loop(0, WIN)
    def _(r):
        for q in range(D // L):
            gbuf[0, r, pl.ds(q * L, L)] = zvec

    zbase = sub * ZROWS

    @pl.loop(0, ZROWS // WIN)
    def _(i):
        pltpu.async_copy(gbuf.at[0], acc.at[pl.ds(zbase + i * WIN, WIN)], sem_i)

    rem = ZROWS % WIN
    if rem:
        pltpu.async_copy(gbuf.at[0, pl.ds(0, rem)],
                         acc.at[pl.ds(zbase + (ZROWS // WIN) * WIN, rem)], sem_i)

    def _bs_body(_, lohi):
        lo, hi = lohi
        mid = (lo + hi) // 2
        a16 = (mid // L) * L
        pltpu.sync_copy(row_hbm.at[pl.ds(a16, L)], bsv)
        probe = bsv[...]
        x = jnp.sum(jnp.where(iot == (mid - a16), probe, jnp.zeros_like(probe)))
        take_hi = x >= HALF
        return (jnp.where(take_hi, lo, mid + 1), jnp.where(take_hi, mid, hi))

    m, _ = lax.fori_loop(0, 20, _bs_body, (jnp.int32(0), jnp.int32(E)))

    @pl.loop(0, ZROWS // WIN)
    def _(i):
        pltpu.make_async_copy(gbuf.at[0],
                              acc.at[pl.ds(zbase + i * WIN, WIN)], sem_i).wait()
    if ZROWS % WIN:
        pltpu.make_async_copy(
            gbuf.at[0, pl.ds(0, ZROWS % WIN)],
            acc.at[pl.ds(zbase + (ZROWS // WIN) * WIN, ZROWS % WIN)],
            sem_i).wait()

    lo_c = jnp.where(core == 0, 0, m)
    hi_c = jnp.where(core == 0, m, E)
    cnt = hi_c - lo_c
    b_start = lo_c + (cnt * sub) // NS
    b_end = lo_c + (cnt * (sub + 1)) // NS
    row_base = core * HALF

    plsc.subcore_barrier()

    abase = (b_start // 8) * 8
    nwin = jnp.maximum((b_end - abase + (WIN - 1)) // WIN, 1)

    def _astart(w):
        return jnp.minimum(abase + w * WIN, E - WIN)

    def _issue_idx(w, b):
        a = _astart(w)
        pltpu.async_copy(col_hbm.at[pl.ds(a, WIN)], colv.at[b], sem_i)
        pltpu.async_copy(row_hbm.at[pl.ds(a, WIN)], rowv.at[b], sem_i)
        pltpu.async_copy(val_hbm.at[pl.ds(a, WIN)], valv.at[b], sem_i)

    def _wait_idx(b):
        pltpu.make_async_copy(col_hbm.at[pl.ds(0, WIN)], colv.at[b], sem_i).wait()
        pltpu.make_async_copy(row_hbm.at[pl.ds(0, WIN)], rowv.at[b], sem_i).wait()
        pltpu.make_async_copy(val_hbm.at[pl.ds(0, WIN)], valv.at[b], sem_i).wait()

    def _issue_gather(b):
        pltpu.async_copy(ego_hbm.at[colv.at[b]], gbuf.at[b], sem)

    def _wait_gather(b):
        pltpu.make_async_copy(ego_hbm.at[colv.at[b]], gbuf.at[b], sem).wait()

    for _k in range(2):
        @pl.when(_k < nwin)
        def _(k=_k):
            _issue_idx(k, k)
    for _k in range(2):
        @pl.when(_k < nwin)
        def _(k=_k):
            _wait_idx(k)
            _issue_gather(k)

    trashv = jnp.full((L,), TRASH, jnp.int32) + iot

    def _window(w, carry):
        b = w % 3
        b2 = (w + 2) % 3

        @pl.when(w + 2 < nwin)
        def _():
            _issue_idx(w + 2, b2)

        _wait_gather(b)

        @pl.when(w + 2 < nwin)
        def _():
            _wait_idx(b2)
            @pl.when(w >= 1)
            def _():
                pltpu.make_async_copy(gbuf.at[b2], acc.at[rowlv.at[b2]],
                                      sem_s).wait()
            _issue_gather(b2)

        nom = abase + w * WIN
        a = _astart(w)
        vlo = jnp.maximum(nom, b_start)
        vhi = b_end

        @plsc.parallel_loop(0, WIN // L, unroll=2)
        def _(g):
            eoff = a + g * L
            idxe = eoff + iot
            valid = (idxe >= vlo) & (idxe < vhi)
            rw = rowv[b, pl.ds(g * L, L)]
            rlocal = jnp.where(valid, rw - row_base, trashv)
            rowlv[b, pl.ds(g * L, L)] = rlocal
            vw = valv[b, pl.ds(g * L, L)]
            for j in range(L):
                er = g * L + j
                bb = _bcast_lane(vw, j)
                for q in range(D // L):
                    sl = pl.ds(q * L, L)
                    gbuf[b, er, sl] = gbuf[b, er, sl] * bb

        pltpu.async_copy(gbuf.at[b], acc.at[rowlv.at[b]], sem_s, add=True)
        return carry

    lax.fori_loop(0, nwin, _window, 0)

    @pl.when(nwin > 2)
    def _():
        pltpu.make_async_copy(gbuf.at[0], acc.at[rowlv.at[0]], sem_s).wait()

    @pl.when(nwin > 1)
    def _():
        pltpu.make_async_copy(gbuf.at[0], acc.at[rowlv.at[0]], sem_s).wait()

    pltpu.make_async_copy(gbuf.at[0], acc.at[rowlv.at[0]], sem_s).wait()

    plsc.subcore_barrier()

    main = (HALF // NS // 8) * 8
    pltpu.sync_copy(acc.at[pl.ds(sub * main, main)],
                    out_hbm.at[pl.ds(row_base + sub * main, main)])
    tail = HALF - NS * main

    @pl.when(sub < tail)
    def _():
        pltpu.sync_copy(acc.at[pl.ds(NS * main + sub, 1)],
                        out_hbm.at[pl.ds(row_base + NS * main + sub, 1)])


def _dense_body(side_ref, ego_ref, wg_ref, bg_ref, wb_ref, bb_ref,
                ego_out_ref, norm_out_ref):
    s = side_ref[...]
    e = ego_ref[...]
    x = jnp.dot(s, wg_ref[...], preferred_element_type=jnp.float32) + bg_ref[...]
    x = x + jnp.dot(e * s, wb_ref[...], preferred_element_type=jnp.float32) + bb_ref[...]
    y = jnp.where(x >= 0, x, x * jnp.float32(0.2))
    ego_out_ref[...] = y
    nrm = jnp.sqrt(jnp.sum(y * y, axis=1, keepdims=True))
    norm_out_ref[...] = y / jnp.maximum(nrm, jnp.float32(1e-12))


_R = 2000
_dense = pl.pallas_call(
    _dense_body,
    grid=(N // _R,),
    in_specs=[
        pl.BlockSpec((_R, D), lambda i: (i, 0)),
        pl.BlockSpec((_R, D), lambda i: (i, 0)),
        pl.BlockSpec((D, D), lambda i: (0, 0)),
        pl.BlockSpec((1, D), lambda i: (0, 0)),
        pl.BlockSpec((D, D), lambda i: (0, 0)),
        pl.BlockSpec((1, D), lambda i: (0, 0)),
    ],
    out_specs=[
        pl.BlockSpec((_R, D), lambda i: (i, 0)),
        pl.BlockSpec((_R, D), lambda i: (i, 0)),
    ],
    out_shape=[
        jax.ShapeDtypeStruct((N, D), jnp.float32),
        jax.ShapeDtypeStruct((N, D), jnp.float32),
    ],
)

_GW = 3 * B // NW


@functools.partial(
    pl.kernel,
    mesh=_mesh,
    compiler_params=_sc_params,
    out_type=jax.ShapeDtypeStruct((4, 3 * B, D), jnp.float32),
    scratch_types=[
        pltpu.VMEM((_GW,), jnp.int32),
        pltpu.VMEM((_GW, D), jnp.float32),
        pltpu.SemaphoreType.DMA,
    ],
)
def _gather4(e0, e1, e2, e3, idx_hbm, out_hbm, idxv, gv, sem):
    wid = lax.axis_index("s") * NC + lax.axis_index("c")
    base = wid * _GW
    pltpu.sync_copy(idx_hbm.at[pl.ds(base, _GW)], idxv)
    for k, eref in enumerate((e0, e1, e2, e3)):
        pltpu.async_copy(eref.at[idxv], gv, sem).wait()
        pltpu.sync_copy(gv, out_hbm.at[k, pl.ds(base, _GW)])


def kernel(user_emb, item_emb,
           W_gc_0, b_gc_0, W_bi_0, b_bi_0,
           W_gc_1, b_gc_1, W_bi_1, b_bi_1,
           W_gc_2, b_gc_2, W_bi_2, b_bi_2,
           adj_row, adj_col, adj_val,
           users, pos_items, neg_items):
    ego = jnp.concatenate([user_emb, item_emb], axis=0)
    col = adj_col.astype(jnp.int32)
    row = adj_row.astype(jnp.int32)
    val = adj_val.astype(jnp.float32)

    embs = [ego]
    for (wg, bg, wb, bb) in (
        (W_gc_0, b_gc_0, W_bi_0, b_bi_0),
        (W_gc_1, b_gc_1, W_bi_1, b_bi_1),
        (W_gc_2, b_gc_2, W_bi_2, b_bi_2),
    ):
        side = _spmm(ego, col, row, val)
        ego, nrm = _dense(side, ego, wg, bg, wb, bb)
        embs.append(nrm)

    idx = jnp.concatenate([
        users.astype(jnp.int32),
        pos_items.astype(jnp.int32) + N_USER,
        neg_items.astype(jnp.int32) + N_USER,
    ])
    g = _gather4(embs[0], embs[1], embs[2], embs[3], idx)
    allg = jnp.transpose(g, (1, 0, 2)).reshape(3 * B, 4 * D)
    return (allg[:B], allg[B:2 * B], allg[2 * B:])


# speedup vs baseline: 1.0622x; 1.0001x over previous
import dataclasses
import functools

import jax
import jax.numpy as jnp
from jax import lax
from jax.experimental import pallas as pl
from jax.experimental.pallas import tpu as pltpu
from jax.experimental.pallas import tpu_sc as plsc

N_USER = 10000
N_ITEM = 40000
N = N_USER + N_ITEM
E = 800000
D = 64
B = 1024

NC = 2
NS = 16
NW = NC * NS
L = 16

HALF = N // 2
ACC_ROWS = 25024
TRASH = 25008
ZROWS = 1564
WIN = 128

_mesh = plsc.VectorSubcoreMesh(core_axis_name="c", subcore_axis_name="s")

_sc_params = pltpu.CompilerParams()
for _f, _v in (("needs_layout_passes", False), ("use_tc_tiling_on_sc", False)):
    if _f in pltpu.CompilerParams.__dataclass_fields__:
        _sc_params = dataclasses.replace(_sc_params, **{_f: _v})


_GDN = lax.GatherDimensionNumbers(
    offset_dims=(), collapsed_slice_dims=(0,), start_index_map=(0,))


def _bcast_lane(vec, j):
    idx = jnp.full((L, 1), j, jnp.int32)
    return lax.gather(vec, idx, _GDN, (1,),
                      mode=lax.GatherScatterMode.PROMISE_IN_BOUNDS)


@functools.partial(
    pl.kernel,
    mesh=_mesh,
    compiler_params=_sc_params,
    out_type=jax.ShapeDtypeStruct((N, D), jnp.float32),
    scratch_types=[
        pltpu.VMEM((3, WIN), jnp.int32),
        pltpu.VMEM((3, WIN), jnp.int32),
        pltpu.VMEM((3, WIN), jnp.float32),
        pltpu.VMEM((3, WIN), jnp.int32),
        pltpu.VMEM((3, WIN, D), jnp.float32),
        pltpu.VMEM((L,), jnp.int32),
        pltpu.VMEM_SHARED((ACC_ROWS, D), jnp.float32),
        pltpu.SemaphoreType.DMA,
        pltpu.SemaphoreType.DMA,
        pltpu.SemaphoreType.DMA,
    ],
)
def _spmm(ego_hbm, col_hbm, row_hbm, val_hbm, out_hbm,
          colv, rowv, valv, rowlv, gbuf, bsv, acc, sem, sem_i, sem_s):
    core = lax.axis_index("c")
    sub = lax.axis_index("s")
    iot = lax.iota(jnp.int32, L)
    zvec = jnp.zeros((L,), jnp.float32)

    @pl.loop(0, WIN)
    def _(r):
        for q in range(D // L):
            gbuf[0, r, pl.ds(q * L, L)] = zvec

    zbase = sub * ZROWS

    @pl.loop(0, ZROWS // WIN)
    def _(i):
        pltpu.async_copy(gbuf.at[0], acc.at[pl.ds(zbase + i * WIN, WIN)], sem_i)

    rem = ZROWS % WIN
    if rem:
        pltpu.async_copy(gbuf.at[0, pl.ds(0, rem)],
                         acc.at[pl.ds(zbase + (ZROWS // WIN) * WIN, rem)], sem_i)

    def _bs_body(_, lohi):
        lo, hi = lohi
        mid = (lo + hi) // 2
        a16 = (mid // L) * L
        pltpu.sync_copy(row_hbm.at[pl.ds(a16, L)], bsv)
        probe = bsv[...]
        x = jnp.sum(jnp.where(iot == (mid - a16), probe, jnp.zeros_like(probe)))
        take_hi = x >= HALF
        return (jnp.where(take_hi, lo, mid + 1), jnp.where(take_hi, mid, hi))

    m, _ = lax.fori_loop(0, 20, _bs_body, (jnp.int32(0), jnp.int32(E)))

    @pl.loop(0, ZROWS // WIN)
    def _(i):
        pltpu.make_async_copy(gbuf.at[0],
                              acc.at[pl.ds(zbase + i * WIN, WIN)], sem_i).wait()
    if ZROWS % WIN:
        pltpu.make_async_copy(
            gbuf.at[0, pl.ds(0, ZROWS % WIN)],
            acc.at[pl.ds(zbase + (ZROWS // WIN) * WIN, ZROWS % WIN)],
            sem_i).wait()

    lo_c = jnp.where(core == 0, 0, m)
    hi_c = jnp.where(core == 0, m, E)
    cnt = hi_c - lo_c
    b_start = lo_c + (cnt * sub) // NS
    b_end = lo_c + (cnt * (sub + 1)) // NS
    row_base = core * HALF

    plsc.subcore_barrier()

    abase = (b_start // 8) * 8
    nwin = jnp.maximum((b_end - abase + (WIN - 1)) // WIN, 1)

    def _astart(w):
        return jnp.minimum(abase + w * WIN, E - WIN)

    def _issue_idx(w, b):
        a = _astart(w)
        pltpu.async_copy(col_hbm.at[pl.ds(a, WIN)], colv.at[b], sem_i)
        pltpu.async_copy(row_hbm.at[pl.ds(a, WIN)], rowv.at[b], sem_i)
        pltpu.async_copy(val_hbm.at[pl.ds(a, WIN)], valv.at[b], sem_i)

    def _wait_idx(b):
        pltpu.make_async_copy(col_hbm.at[pl.ds(0, WIN)], colv.at[b], sem_i).wait()
        pltpu.make_async_copy(row_hbm.at[pl.ds(0, WIN)], rowv.at[b], sem_i).wait()
        pltpu.make_async_copy(val_hbm.at[pl.ds(0, WIN)], valv.at[b], sem_i).wait()

    def _issue_gather(b):
        pltpu.async_copy(ego_hbm.at[colv.at[b]], gbuf.at[b], sem)

    def _wait_gather(b):
        pltpu.make_async_copy(ego_hbm.at[colv.at[b]], gbuf.at[b], sem).wait()

    for _k in range(2):
        @pl.when(_k < nwin)
        def _(k=_k):
            _issue_idx(k, k)
    for _k in range(2):
        @pl.when(_k < nwin)
        def _(k=_k):
            _wait_idx(k)
            _issue_gather(k)

    trashv = jnp.full((L,), TRASH, jnp.int32) + iot

    def _window(w, carry):
        b = w % 3
        b2 = (w + 2) % 3

        @pl.when(w + 2 < nwin)
        def _():
            _issue_idx(w + 2, b2)

        _wait_gather(b)

        @pl.when(w + 2 < nwin)
        def _():
            _wait_idx(b2)
            @pl.when(w >= 1)
            def _():
                pltpu.make_async_copy(gbuf.at[b2], acc.at[rowlv.at[b2]],
                                      sem_s).wait()
            _issue_gather(b2)

        nom = abase + w * WIN
        a = _astart(w)
        vlo = jnp.maximum(nom, b_start)
        vhi = b_end

        @plsc.parallel_loop(0, WIN // L, unroll=2)
        def _(g):
            eoff = a + g * L
            idxe = eoff + iot
            valid = (idxe >= vlo) & (idxe < vhi)
            rw = rowv[b, pl.ds(g * L, L)]
            rlocal = jnp.where(valid, rw - row_base, trashv)
            rowlv[b, pl.ds(g * L, L)] = rlocal
            vw = valv[b, pl.ds(g * L, L)]
            for j in range(L):
                er = g * L + j
                bb = _bcast_lane(vw, j)
                for q in range(D // L):
                    sl = pl.ds(q * L, L)
                    gbuf[b, er, sl] = gbuf[b, er, sl] * bb

        pltpu.async_copy(gbuf.at[b], acc.at[rowlv.at[b]], sem_s, add=True)
        return carry

    lax.fori_loop(0, nwin, _window, 0)

    @pl.when(nwin > 2)
    def _():
        pltpu.make_async_copy(gbuf.at[0], acc.at[rowlv.at[0]], sem_s).wait()

    @pl.when(nwin > 1)
    def _():
        pltpu.make_async_copy(gbuf.at[0], acc.at[rowlv.at[0]], sem_s).wait()

    pltpu.make_async_copy(gbuf.at[0], acc.at[rowlv.at[0]], sem_s).wait()

    plsc.subcore_barrier()

    main = (HALF // NS // 8) * 8
    pltpu.sync_copy(acc.at[pl.ds(sub * main, main)],
                    out_hbm.at[pl.ds(row_base + sub * main, main)])
    tail = HALF - NS * main

    @pl.when(sub < tail)
    def _():
        pltpu.sync_copy(acc.at[pl.ds(NS * main + sub, 1)],
                        out_hbm.at[pl.ds(row_base + NS * main + sub, 1)])


def _dense_body(side_ref, ego_ref, wg_ref, bg_ref, wb_ref, bb_ref,
                ego_out_ref, norm_out_ref):
    s = side_ref[...]
    e = ego_ref[...]
    x = jnp.dot(s, wg_ref[...], preferred_element_type=jnp.float32) + bg_ref[...]
    x = x + jnp.dot(e * s, wb_ref[...], preferred_element_type=jnp.float32) + bb_ref[...]
    y = jnp.where(x >= 0, x, x * jnp.float32(0.2))
    ego_out_ref[...] = y
    nrm = jnp.sqrt(jnp.sum(y * y, axis=1, keepdims=True))
    norm_out_ref[...] = y / jnp.maximum(nrm, jnp.float32(1e-12))


_R = 2000
_dense = pl.pallas_call(
    _dense_body,
    grid=(N // _R,),
    in_specs=[
        pl.BlockSpec((_R, D), lambda i: (i, 0)),
        pl.BlockSpec((_R, D), lambda i: (i, 0)),
        pl.BlockSpec((D, D), lambda i: (0, 0)),
        pl.BlockSpec((1, D), lambda i: (0, 0)),
        pl.BlockSpec((D, D), lambda i: (0, 0)),
        pl.BlockSpec((1, D), lambda i: (0, 0)),
    ],
    out_specs=[
        pl.BlockSpec((_R, D), lambda i: (i, 0)),
        pl.BlockSpec((_R, D), lambda i: (i, 0)),
    ],
    out_shape=[
        jax.ShapeDtypeStruct((N, D), jnp.float32),
        jax.ShapeDtypeStruct((N, D), jnp.float32),
    ],
)

_GW = 3 * B // NW


@functools.partial(
    pl.kernel,
    mesh=_mesh,
    compiler_params=_sc_params,
    out_type=jax.ShapeDtypeStruct((4, 3 * B, D), jnp.float32),
    scratch_types=[
        pltpu.VMEM((_GW,), jnp.int32),
        pltpu.VMEM((_GW, D), jnp.float32),
        pltpu.SemaphoreType.DMA,
    ],
)
def _gather4(e0, e1, e2, e3, idx_hbm, out_hbm, idxv, gv, sem):
    wid = lax.axis_index("s") * NC + lax.axis_index("c")
    base = wid * _GW
    pltpu.sync_copy(idx_hbm.at[pl.ds(base, _GW)], idxv)
    for k, eref in enumerate((e0, e1, e2, e3)):
        pltpu.async_copy(eref.at[idxv], gv, sem).wait()
        pltpu.sync_copy(gv, out_hbm.at[k, pl.ds(base, _GW)])


def kernel(user_emb, item_emb,
           W_gc_0, b_gc_0, W_bi_0, b_bi_0,
           W_gc_1, b_gc_1, W_bi_1, b_bi_1,
           W_gc_2, b_gc_2, W_bi_2, b_bi_2,
           adj_row, adj_col, adj_val,
           users, pos_items, neg_items):
    ego = jnp.concatenate([user_emb, item_emb], axis=0)
    col = adj_col.astype(jnp.int32)
    row = adj_row.astype(jnp.int32)
    val = adj_val.astype(jnp.float32)

    embs = [ego]
    for (wg, bg, wb, bb) in (
        (W_gc_0, b_gc_0, W_bi_0, b_bi_0),
        (W_gc_1, b_gc_1, W_bi_1, b_bi_1),
        (W_gc_2, b_gc_2, W_bi_2, b_bi_2),
    ):
        side = _spmm(ego, col, row, val)
        ego, nrm = _dense(side, ego, wg, bg, wb, bb)
        embs.append(nrm)

    idx = jnp.concatenate([
        users.astype(jnp.int32),
        pos_items.astype(jnp.int32) + N_USER,
        neg_items.astype(jnp.int32) + N_USER,
    ])
    g = _gather4(embs[0], embs[1], embs[2], embs[3], idx)
    allg = jnp.transpose(g, (1, 0, 2)).reshape(3 * B, 4 * D)
    return (allg[:B], allg[B:2 * B], allg[2 * B:])
